# initial kernel scaffold (unmeasured)
import jax
import jax.numpy as jnp
from jax import lax
from jax.experimental import pallas as pl
from jax.experimental.pallas import tpu as pltpu

N_DEV = 32


def kernel(x, w_mat, scale_x, scale_w):
    m_per, k = x.shape
    _, n_per = w_mat.shape

    def body(x_ref, w_ref, sx_ref, sw_ref, out_ref,
             gx_ref, w_bf_ref, send_sem, recv_sems):
        my = lax.axis_index("i")
        left = lax.rem(my + N_DEV - 1, N_DEV)
        right = lax.rem(my + 1, N_DEV)

        barrier_sem = pltpu.get_barrier_semaphore()
        for nbr in (left, right):
            pl.semaphore_signal(barrier_sem, inc=1, device_id=(nbr,),
                                device_id_type=pl.DeviceIdType.MESH)
        pl.semaphore_wait(barrier_sem, 2)

        w_bf_ref[...] = w_ref[...].astype(jnp.bfloat16)
        s = sx_ref[0] * sw_ref[0]

        gx_ref[my] = x_ref[...]

        def compute(origin):
            a = gx_ref[origin].astype(jnp.bfloat16)
            acc = jnp.dot(a, w_bf_ref[...], preferred_element_type=jnp.float32)
            out_ref[pl.ds(origin * m_per, m_per), :] = jnp.maximum(acc * s, 0.0)

        compute(my)

        for h in range(N_DEV - 1):
            src_origin = lax.rem(my - h + N_DEV, N_DEV)
            rdma = pltpu.make_async_remote_copy(
                src_ref=gx_ref.at[src_origin],
                dst_ref=gx_ref.at[src_origin],
                send_sem=send_sem,
                recv_sem=recv_sems.at[h],
                device_id=(right,),
                device_id_type=pl.DeviceIdType.MESH,
            )
            rdma.start()
            rdma.wait()
            recv_origin = lax.rem(my - h - 1 + N_DEV, N_DEV)
            compute(recv_origin)

    return pl.pallas_call(
        body,
        out_shape=jax.ShapeDtypeStruct((N_DEV * m_per, n_per), jnp.float32),
        in_specs=[
            pl.BlockSpec(memory_space=pltpu.VMEM),
            pl.BlockSpec(memory_space=pltpu.VMEM),
            pl.BlockSpec(memory_space=pltpu.SMEM),
            pl.BlockSpec(memory_space=pltpu.SMEM),
        ],
        out_specs=pl.BlockSpec(memory_space=pltpu.VMEM),
        scratch_shapes=[
            pltpu.VMEM((N_DEV, m_per, k), x.dtype),
            pltpu.VMEM((k, n_per), jnp.bfloat16),
            pltpu.SemaphoreType.DMA,
            pltpu.SemaphoreType.DMA((N_DEV - 1,)),
        ],
        compiler_params=pltpu.CompilerParams(collective_id=0),
    )(x, w_mat, scale_x, scale_w)


# baseline (device time: 264723 ns/iter reference)
import jax
import jax.numpy as jnp
from jax import lax
from jax.experimental import pallas as pl
from jax.experimental.pallas import tpu as pltpu

N_DEV = 32


def kernel(x, w_mat, scale_x, scale_w):
    m_per, k = x.shape
    _, n_per = w_mat.shape

    x = x.astype(jnp.float8_e4m3fn)
    w_mat = w_mat.astype(jnp.bfloat16)

    def body(x_ref, w_ref, sx_ref, sw_ref, out_ref,
             gx_ref, send_sem, recv_sems):
        my = lax.axis_index("i")
        left = lax.rem(my + N_DEV - 1, N_DEV)
        right = lax.rem(my + 1, N_DEV)

        barrier_sem = pltpu.get_barrier_semaphore()
        for nbr in (left, right):
            pl.semaphore_signal(barrier_sem, inc=1, device_id=(nbr,),
                                device_id_type=pl.DeviceIdType.MESH)
        pl.semaphore_wait(barrier_sem, 2)

        s = sx_ref[0] * sw_ref[0]

        gx_ref[my] = x_ref[...]

        def compute(origin):
            a = gx_ref[origin].astype(jnp.bfloat16)
            acc = jnp.dot(a, w_ref[...], preferred_element_type=jnp.float32)
            out_ref[pl.ds(origin * m_per, m_per), :] = jnp.maximum(acc * s, 0.0)

        compute(my)

        for h in range(N_DEV - 1):
            src_origin = lax.rem(my - h + N_DEV, N_DEV)
            rdma = pltpu.make_async_remote_copy(
                src_ref=gx_ref.at[src_origin],
                dst_ref=gx_ref.at[src_origin],
                send_sem=send_sem,
                recv_sem=recv_sems.at[h],
                device_id=(right,),
                device_id_type=pl.DeviceIdType.MESH,
            )
            rdma.start()
            rdma.wait()
            recv_origin = lax.rem(my - h - 1 + N_DEV, N_DEV)
            compute(recv_origin)

    return pl.pallas_call(
        body,
        out_shape=jax.ShapeDtypeStruct((N_DEV * m_per, n_per), jnp.float32),
        in_specs=[
            pl.BlockSpec(memory_space=pltpu.VMEM),
            pl.BlockSpec(memory_space=pltpu.VMEM),
            pl.BlockSpec(memory_space=pltpu.SMEM),
            pl.BlockSpec(memory_space=pltpu.SMEM),
        ],
        out_specs=pl.BlockSpec(memory_space=pltpu.VMEM),
        scratch_shapes=[
            pltpu.VMEM((N_DEV, m_per, k), x.dtype),
            pltpu.SemaphoreType.DMA,
            pltpu.SemaphoreType.DMA((N_DEV - 1,)),
        ],
        compiler_params=pltpu.CompilerParams(collective_id=0),
    )(x, w_mat, scale_x, scale_w)


# device time: 192984 ns/iter; 1.3717x vs baseline; 1.3717x over previous
import jax
import jax.numpy as jnp
from jax import lax
from jax.experimental import pallas as pl
from jax.experimental.pallas import tpu as pltpu

N_DEV = 32
R_HOPS = 16
L_HOPS = 15


def kernel(x, w_mat, scale_x, scale_w):
    m_per, k = x.shape
    _, n_per = w_mat.shape

    x = x.astype(jnp.float8_e4m3fn)
    w_mat = w_mat.astype(jnp.bfloat16)

    def body(x_ref, w_ref, sx_ref, sw_ref, out_ref,
             gx_ref, rsend_sems, lsend_sems, larecv_sems, rarecv_sems):
        my = lax.axis_index("i")
        left = lax.rem(my + N_DEV - 1, N_DEV)
        right = lax.rem(my + 1, N_DEV)

        barrier_sem = pltpu.get_barrier_semaphore()
        for nbr in (left, right):
            pl.semaphore_signal(barrier_sem, inc=1, device_id=(nbr,),
                                device_id_type=pl.DeviceIdType.MESH)
        pl.semaphore_wait(barrier_sem, 2)

        s = sx_ref[0] * sw_ref[0]
        gx_ref[my] = x_ref[...]

        def compute(origin):
            a = gx_ref[origin].astype(jnp.bfloat16)
            acc = jnp.dot(a, w_ref[...], preferred_element_type=jnp.float32)
            out_ref[pl.ds(origin * m_per, m_per), :] = jnp.maximum(acc * s, 0.0)

        def send_right(h):
            origin = lax.rem(my - h + N_DEV, N_DEV)
            rdma = pltpu.make_async_remote_copy(
                src_ref=gx_ref.at[origin], dst_ref=gx_ref.at[origin],
                send_sem=rsend_sems.at[h], recv_sem=larecv_sems.at[h],
                device_id=(right,), device_id_type=pl.DeviceIdType.MESH,
            )
            rdma.start()
            return rdma

        def send_left(h):
            origin = lax.rem(my + h, N_DEV)
            rdma = pltpu.make_async_remote_copy(
                src_ref=gx_ref.at[origin], dst_ref=gx_ref.at[origin],
                send_sem=lsend_sems.at[h], recv_sem=rarecv_sems.at[h],
                device_id=(left,), device_id_type=pl.DeviceIdType.MESH,
            )
            rdma.start()
            return rdma

        def recv_from_left(h):
            origin = lax.rem(my - h - 1 + N_DEV, N_DEV)
            rdma = pltpu.make_async_remote_copy(
                src_ref=gx_ref.at[origin], dst_ref=gx_ref.at[origin],
                send_sem=rsend_sems.at[h], recv_sem=larecv_sems.at[h],
                device_id=(right,), device_id_type=pl.DeviceIdType.MESH,
            )
            rdma.wait_recv()
            return origin

        def recv_from_right(h):
            origin = lax.rem(my + h + 1, N_DEV)
            rdma = pltpu.make_async_remote_copy(
                src_ref=gx_ref.at[origin], dst_ref=gx_ref.at[origin],
                send_sem=lsend_sems.at[h], recv_sem=rarecv_sems.at[h],
                device_id=(left,), device_id_type=pl.DeviceIdType.MESH,
            )
            rdma.wait_recv()
            return origin

        sends = [send_right(0), send_left(0)]
        compute(my)

        for h in range(R_HOPS):
            o = recv_from_left(h)
            if h + 1 < R_HOPS:
                sends.append(send_right(h + 1))
            compute(o)
            if h < L_HOPS:
                o = recv_from_right(h)
                if h + 1 < L_HOPS:
                    sends.append(send_left(h + 1))
                compute(o)

        for rdma in sends:
            rdma.wait_send()

    return pl.pallas_call(
        body,
        out_shape=jax.ShapeDtypeStruct((N_DEV * m_per, n_per), jnp.float32),
        in_specs=[
            pl.BlockSpec(memory_space=pltpu.VMEM),
            pl.BlockSpec(memory_space=pltpu.VMEM),
            pl.BlockSpec(memory_space=pltpu.SMEM),
            pl.BlockSpec(memory_space=pltpu.SMEM),
        ],
        out_specs=pl.BlockSpec(memory_space=pltpu.VMEM),
        scratch_shapes=[
            pltpu.VMEM((N_DEV, m_per, k), x.dtype),
            pltpu.SemaphoreType.DMA((R_HOPS,)),
            pltpu.SemaphoreType.DMA((L_HOPS,)),
            pltpu.SemaphoreType.DMA((R_HOPS,)),
            pltpu.SemaphoreType.DMA((L_HOPS,)),
        ],
        compiler_params=pltpu.CompilerParams(collective_id=0),
    )(x, w_mat, scale_x, scale_w)


# device time: 128778 ns/iter; 2.0557x vs baseline; 1.4986x over previous
import jax
import jax.numpy as jnp
from jax import lax
from jax.experimental import pallas as pl
from jax.experimental.pallas import tpu as pltpu

N_DEV = 32
R_HOPS = 16
L_HOPS = 15


def _ring_pos(idx):
    z = idx // 8
    r = idx % 8
    y = r // 2
    xb = r % 2
    x = jnp.where(y % 2 == 1, 1 - xb, xb)
    q = z * 4 + jnp.where(z % 2 == 1, 3 - y, y)
    return jnp.where(x == 0, q, 31 - q)


def _mesh_idx(rp):
    rp = lax.rem(rp + N_DEV, N_DEV)
    on_x0 = rp < 16
    q = jnp.where(on_x0, rp, 31 - rp)
    x = jnp.where(on_x0, 0, 1)
    z = q // 4
    yq = q % 4
    y = jnp.where(z % 2 == 1, 3 - yq, yq)
    xb = jnp.where(y % 2 == 1, 1 - x, x)
    return z * 8 + y * 2 + xb


def kernel(x, w_mat, scale_x, scale_w):
    m_per, k = x.shape
    _, n_per = w_mat.shape

    x = x.astype(jnp.float8_e4m3fn)
    w_mat = w_mat.astype(jnp.bfloat16)

    def body(x_ref, w_ref, sx_ref, sw_ref, out_ref,
             gx_ref, rsend_sems, lsend_sems, larecv_sems, rarecv_sems):
        my = lax.axis_index("i")
        rp = _ring_pos(my)
        left = _mesh_idx(rp - 1)
        right = _mesh_idx(rp + 1)

        barrier_sem = pltpu.get_barrier_semaphore()
        for nbr in (left, right):
            pl.semaphore_signal(barrier_sem, inc=1, device_id=(nbr,),
                                device_id_type=pl.DeviceIdType.MESH)
        pl.semaphore_wait(barrier_sem, 2)

        s = sx_ref[0] * sw_ref[0]
        gx_ref[my] = x_ref[...]

        def compute(origin):
            a = gx_ref[origin].astype(jnp.bfloat16)
            acc = jnp.dot(a, w_ref[...], preferred_element_type=jnp.float32)
            out_ref[pl.ds(origin * m_per, m_per), :] = jnp.maximum(acc * s, 0.0)

        def send_right(h):
            origin = _mesh_idx(rp - h)
            rdma = pltpu.make_async_remote_copy(
                src_ref=gx_ref.at[origin], dst_ref=gx_ref.at[origin],
                send_sem=rsend_sems.at[h], recv_sem=larecv_sems.at[h],
                device_id=(right,), device_id_type=pl.DeviceIdType.MESH,
            )
            rdma.start()
            return rdma

        def send_left(h):
            origin = _mesh_idx(rp + h)
            rdma = pltpu.make_async_remote_copy(
                src_ref=gx_ref.at[origin], dst_ref=gx_ref.at[origin],
                send_sem=lsend_sems.at[h], recv_sem=rarecv_sems.at[h],
                device_id=(left,), device_id_type=pl.DeviceIdType.MESH,
            )
            rdma.start()
            return rdma

        def recv_from_left(h):
            origin = _mesh_idx(rp - h - 1)
            rdma = pltpu.make_async_remote_copy(
                src_ref=gx_ref.at[origin], dst_ref=gx_ref.at[origin],
                send_sem=rsend_sems.at[h], recv_sem=larecv_sems.at[h],
                device_id=(right,), device_id_type=pl.DeviceIdType.MESH,
            )
            rdma.wait_recv()
            return origin

        def recv_from_right(h):
            origin = _mesh_idx(rp + h + 1)
            rdma = pltpu.make_async_remote_copy(
                src_ref=gx_ref.at[origin], dst_ref=gx_ref.at[origin],
                send_sem=lsend_sems.at[h], recv_sem=rarecv_sems.at[h],
                device_id=(left,), device_id_type=pl.DeviceIdType.MESH,
            )
            rdma.wait_recv()
            return origin

        sends = [send_right(0), send_left(0)]
        compute(my)

        for h in range(R_HOPS):
            o1 = recv_from_left(h)
            if h + 1 < R_HOPS:
                sends.append(send_right(h + 1))
            if h < L_HOPS:
                o2 = recv_from_right(h)
                if h + 1 < L_HOPS:
                    sends.append(send_left(h + 1))
                compute(o1)
                compute(o2)
            else:
                compute(o1)

        for rdma in sends:
            rdma.wait_send()

    return pl.pallas_call(
        body,
        out_shape=jax.ShapeDtypeStruct((N_DEV * m_per, n_per), jnp.float32),
        in_specs=[
            pl.BlockSpec(memory_space=pltpu.VMEM),
            pl.BlockSpec(memory_space=pltpu.VMEM),
            pl.BlockSpec(memory_space=pltpu.SMEM),
            pl.BlockSpec(memory_space=pltpu.SMEM),
        ],
        out_specs=pl.BlockSpec(memory_space=pltpu.VMEM),
        scratch_shapes=[
            pltpu.VMEM((N_DEV, m_per, k), x.dtype),
            pltpu.SemaphoreType.DMA((R_HOPS,)),
            pltpu.SemaphoreType.DMA((L_HOPS,)),
            pltpu.SemaphoreType.DMA((R_HOPS,)),
            pltpu.SemaphoreType.DMA((L_HOPS,)),
        ],
        compiler_params=pltpu.CompilerParams(collective_id=0),
    )(x, w_mat, scale_x, scale_w)
